# Initial kernel scaffold; baseline (speedup 1.0000x reference)
import jax
import jax.numpy as jnp
from jax.experimental import pallas as pl


def kernel(ray_dirs, log_depth, control_points):
    raise NotImplementedError("write your pallas kernel here")


# speedup vs baseline: 116.2619x; 116.2619x over previous
import functools

import jax
import jax.numpy as jnp
from jax import lax
from jax.experimental import pallas as pl
from jax.experimental.pallas import tpu as pltpu
from jax.experimental.pallas import tpu_sc as plsc

N_ALPHA = 16
N_DEPTH = 12
ALPHA_MIN = -1.5707963267948966
ALPHA_RANGE = 3.141592653589793 + 1e-8
LD_MIN = -3.0
LD_RANGE = 8.0 + 1e-8
MAX_DELTA = 0.5

NUM_WORKERS = 32
LANES = 16

_ASIN_C = (
    1.5707963050,
    -0.2145988016,
    0.0889789874,
    -0.0501743046,
    0.0308918810,
    -0.0170881256,
    0.0066700901,
    -0.0012624911,
)


def _vsqrt(u):
    i = plsc.bitcast(u, jnp.int32)
    i = jnp.int32(0x5F3759DF) - lax.shift_right_logical(i, 1)
    r = plsc.bitcast(i, jnp.float32)
    half_u = 0.5 * u
    for _ in range(3):
        r = r * (1.5 - half_u * r * r)
    return u * r


def _vasin(x):
    t = jnp.abs(x)
    u = jnp.maximum(1.0 - t, 1e-14)
    sq = _vsqrt(u)
    p = jnp.full_like(t, _ASIN_C[7])
    for c in (_ASIN_C[6], _ASIN_C[5], _ASIN_C[4], _ASIN_C[3],
              _ASIN_C[2], _ASIN_C[1], _ASIN_C[0]):
        p = p * t + c
    a = 1.5707963267948966 - sq * p
    a = jnp.where(t >= 1.0, jnp.float32(1.5707963267948966), a)
    return jnp.where(x < 0.0, -a, a)


def _bspline4(u):
    u2 = u * u
    u3 = u2 * u
    omu = 1.0 - u
    b0 = (omu * omu * omu) * (1.0 / 6.0)
    b1 = 0.5 * u3 - u2 + (2.0 / 3.0)
    b3 = u3 * (1.0 / 6.0)
    b2 = 1.0 - b0 - b1 - b3
    return b0, b1, b2, b3


def _make_sc_call(n, chunk):
    rays_per_tile = n // NUM_WORKERS
    chunks_per_tile = rays_per_tile // chunk
    groups_per_chunk = chunk // LANES
    mesh = plsc.VectorSubcoreMesh(core_axis_name="c", subcore_axis_name="s")

    @functools.partial(
        pl.kernel,
        mesh=mesh,
        compiler_params=pltpu.CompilerParams(needs_layout_passes=False),
        out_type=jax.ShapeDtypeStruct((n,), jnp.float32),
        scratch_types=[
            pltpu.VMEM((N_ALPHA * N_DEPTH,), jnp.float32),
            pltpu.VMEM((3 * chunk,), jnp.float32),
            pltpu.VMEM((chunk,), jnp.float32),
            pltpu.VMEM((chunk,), jnp.float32),
        ],
    )
    def sc_call(xyz_hbm, ld_hbm, cp_hbm, out_hbm, table_v, xyz_v, ld_v, out_v):
        wid = lax.axis_index("s") * 2 + lax.axis_index("c")
        base = wid * rays_per_tile
        pltpu.sync_copy(cp_hbm, table_v)

        lane_i32 = lax.iota(jnp.int32, LANES)
        y_sel = lane_i32 * 3 + 1

        def do_chunk(c, carry):
            row0 = base + c * chunk
            pltpu.sync_copy(xyz_hbm.at[pl.ds(row0 * 3, 3 * chunk)], xyz_v)
            pltpu.sync_copy(ld_hbm.at[pl.ds(row0, chunk)], ld_v)

            def do_group(g, carry2):
                off = g * LANES
                y = plsc.load_gather(xyz_v, [y_sel + off * 3])
                ld = ld_v[pl.ds(off, LANES)]

                yc = jnp.clip(y, -1.0, 1.0)
                alpha = _vasin(yc)
                a_norm = jnp.clip((alpha - ALPHA_MIN) / ALPHA_RANGE, 0.0, 1.0)
                d_norm = jnp.clip((ld - LD_MIN) / LD_RANGE, 0.0, 1.0)
                a_idx = a_norm * (N_ALPHA - 1)
                d_idx = d_norm * (N_DEPTH - 1)
                fa = a_idx.astype(jnp.int32)
                fd = d_idx.astype(jnp.int32)
                a_loc = a_idx - fa.astype(jnp.float32)
                d_loc = d_idx - fd.astype(jnp.float32)
                a_start = jnp.clip(fa - 1, 0, N_ALPHA - 4)
                d_start = jnp.clip(fd - 1, 0, N_DEPTH - 4)

                ab = _bspline4(a_loc)
                db = _bspline4(d_loc)

                flat0 = a_start * N_DEPTH + d_start
                acc = None
                for i in range(4):
                    row = None
                    for j in range(4):
                        cv = plsc.load_gather(
                            table_v, [flat0 + (i * N_DEPTH + j)])
                        term = db[j] * cv
                        row = term if row is None else row + term
                    term = ab[i] * row
                    acc = term if acc is None else acc + term

                res = jnp.clip(acc, -MAX_DELTA, MAX_DELTA)
                out_v[pl.ds(off, LANES)] = res
                return carry2

            lax.fori_loop(0, groups_per_chunk, do_group, 0)
            pltpu.sync_copy(out_v, out_hbm.at[pl.ds(row0, chunk)])
            return carry

        lax.fori_loop(0, chunks_per_tile, do_chunk, 0)

    return sc_call


def kernel(ray_dirs, log_depth, control_points):
    n = ray_dirs.shape[0]
    xyz_flat = ray_dirs.reshape(-1)
    cp_flat = control_points.reshape(-1)
    sc_call = _make_sc_call(n, 4096)
    return sc_call(xyz_flat, log_depth, cp_flat)


# speedup vs baseline: 120.6540x; 1.0378x over previous
import functools

import jax
import jax.numpy as jnp
from jax import lax
from jax.experimental import pallas as pl
from jax.experimental.pallas import tpu as pltpu
from jax.experimental.pallas import tpu_sc as plsc

N_ALPHA = 16
N_DEPTH = 12
ALPHA_MIN = -1.5707963267948966
ALPHA_RANGE = 3.141592653589793 + 1e-8
LD_MIN = -3.0
LD_RANGE = 8.0 + 1e-8
MAX_DELTA = 0.5

NUM_WORKERS = 32
LANES = 16

_ASIN_C = (
    1.5707963050,
    -0.2145988016,
    0.0889789874,
    -0.0501743046,
    0.0308918810,
    -0.0170881256,
    0.0066700901,
    -0.0012624911,
)


def _vsqrt(u):
    i = plsc.bitcast(u, jnp.int32)
    i = jnp.int32(0x5F3759DF) - lax.shift_right_logical(i, 1)
    r = plsc.bitcast(i, jnp.float32)
    half_u = 0.5 * u
    for _ in range(3):
        r = r * (1.5 - half_u * r * r)
    return u * r


def _vasin(x):
    t = jnp.abs(x)
    u = jnp.maximum(1.0 - t, 1e-14)
    sq = _vsqrt(u)
    p = jnp.full_like(t, _ASIN_C[7])
    for c in (_ASIN_C[6], _ASIN_C[5], _ASIN_C[4], _ASIN_C[3],
              _ASIN_C[2], _ASIN_C[1], _ASIN_C[0]):
        p = p * t + c
    a = 1.5707963267948966 - sq * p
    a = jnp.where(t >= 1.0, jnp.float32(1.5707963267948966), a)
    return jnp.where(x < 0.0, -a, a)


def _bspline4(u):
    u2 = u * u
    u3 = u2 * u
    omu = 1.0 - u
    b0 = (omu * omu * omu) * (1.0 / 6.0)
    b1 = 0.5 * u3 - u2 + (2.0 / 3.0)
    b3 = u3 * (1.0 / 6.0)
    b2 = 1.0 - b0 - b1 - b3
    return b0, b1, b2, b3


def _make_sc_call(n, chunk):
    rays_per_tile = n // NUM_WORKERS
    chunks_per_tile = rays_per_tile // chunk
    groups_per_chunk = chunk // LANES
    mesh = plsc.VectorSubcoreMesh(core_axis_name="c", subcore_axis_name="s")

    @functools.partial(
        pl.kernel,
        mesh=mesh,
        compiler_params=pltpu.CompilerParams(needs_layout_passes=False),
        out_type=jax.ShapeDtypeStruct((n,), jnp.float32),
        scratch_types=[
            pltpu.VMEM((N_ALPHA * N_DEPTH,), jnp.float32),
            pltpu.VMEM((3 * chunk,), jnp.float32),
            pltpu.VMEM((chunk,), jnp.float32),
            pltpu.VMEM((chunk,), jnp.float32),
        ],
    )
    def sc_call(xyz_hbm, ld_hbm, cp_hbm, out_hbm, table_v, xyz_v, ld_v, out_v):
        wid = lax.axis_index("s") * 2 + lax.axis_index("c")
        base = wid * rays_per_tile
        pltpu.sync_copy(cp_hbm, table_v)

        lane_i32 = lax.iota(jnp.int32, LANES)
        y_sel = lane_i32 * 3 + 1

        def do_chunk(c, carry):
            row0 = base + c * chunk
            pltpu.sync_copy(xyz_hbm.at[pl.ds(row0 * 3, 3 * chunk)], xyz_v)
            pltpu.sync_copy(ld_hbm.at[pl.ds(row0, chunk)], ld_v)

            @plsc.parallel_loop(0, groups_per_chunk, unroll=4)
            def do_group(g):
                off = g * LANES
                y = plsc.load_gather(xyz_v, [y_sel + off * 3])
                ld = ld_v[pl.ds(off, LANES)]

                yc = jnp.clip(y, -1.0, 1.0)
                alpha = _vasin(yc)
                a_norm = jnp.clip((alpha - ALPHA_MIN) / ALPHA_RANGE, 0.0, 1.0)
                d_norm = jnp.clip((ld - LD_MIN) / LD_RANGE, 0.0, 1.0)
                a_idx = a_norm * (N_ALPHA - 1)
                d_idx = d_norm * (N_DEPTH - 1)
                fa = a_idx.astype(jnp.int32)
                fd = d_idx.astype(jnp.int32)
                a_loc = a_idx - fa.astype(jnp.float32)
                d_loc = d_idx - fd.astype(jnp.float32)
                a_start = jnp.clip(fa - 1, 0, N_ALPHA - 4)
                d_start = jnp.clip(fd - 1, 0, N_DEPTH - 4)

                ab = _bspline4(a_loc)
                db = _bspline4(d_loc)

                flat0 = a_start * N_DEPTH + d_start
                acc = None
                for i in range(4):
                    row = None
                    for j in range(4):
                        cv = plsc.load_gather(
                            table_v, [flat0 + (i * N_DEPTH + j)])
                        term = db[j] * cv
                        row = term if row is None else row + term
                    term = ab[i] * row
                    acc = term if acc is None else acc + term

                res = jnp.clip(acc, -MAX_DELTA, MAX_DELTA)
                out_v[pl.ds(off, LANES)] = res

            pltpu.sync_copy(out_v, out_hbm.at[pl.ds(row0, chunk)])
            return carry

        lax.fori_loop(0, chunks_per_tile, do_chunk, 0)

    return sc_call


def kernel(ray_dirs, log_depth, control_points):
    n = ray_dirs.shape[0]
    xyz_flat = ray_dirs.reshape(-1)
    cp_flat = control_points.reshape(-1)
    sc_call = _make_sc_call(n, 4096)
    return sc_call(xyz_flat, log_depth, cp_flat)


# speedup vs baseline: 1917.5216x; 15.8927x over previous
import functools

import jax
import jax.numpy as jnp
from jax import lax
from jax.experimental import pallas as pl
from jax.experimental.pallas import tpu as pltpu
from jax.experimental.pallas import tpu_sc as plsc

N_ALPHA = 16
N_DEPTH = 12
ALPHA_MIN = -1.5707963267948966
ALPHA_RANGE = 3.141592653589793 + 1e-8
LD_MIN = -3.0
LD_RANGE = 8.0 + 1e-8
MAX_DELTA = 0.5

NUM_WORKERS = 32
LANES = 16

_ASIN_C = (
    1.5707963050,
    -0.2145988016,
    0.0889789874,
    -0.0501743046,
    0.0308918810,
    -0.0170881256,
    0.0066700901,
    -0.0012624911,
)


def _vsqrt(u):
    i = plsc.bitcast(u, jnp.int32)
    i = jnp.int32(0x5F3759DF) - lax.shift_right_logical(i, 1)
    r = plsc.bitcast(i, jnp.float32)
    half_u = 0.5 * u
    for _ in range(3):
        r = r * (1.5 - half_u * r * r)
    return u * r


def _vasin(x):
    t = jnp.abs(x)
    u = jnp.maximum(1.0 - t, 1e-14)
    sq = _vsqrt(u)
    p = jnp.full_like(t, _ASIN_C[7])
    for c in (_ASIN_C[6], _ASIN_C[5], _ASIN_C[4], _ASIN_C[3],
              _ASIN_C[2], _ASIN_C[1], _ASIN_C[0]):
        p = p * t + c
    a = 1.5707963267948966 - sq * p
    a = jnp.where(t >= 1.0, jnp.float32(1.5707963267948966), a)
    return jnp.where(x < 0.0, -a, a)


def _bspline4(u):
    u2 = u * u
    u3 = u2 * u
    omu = 1.0 - u
    b0 = (omu * omu * omu) * (1.0 / 6.0)
    b1 = 0.5 * u3 - u2 + (2.0 / 3.0)
    b3 = u3 * (1.0 / 6.0)
    b2 = 1.0 - b0 - b1 - b3
    return b0, b1, b2, b3


def _make_sc_call(n, chunk):
    rays_per_tile = n // NUM_WORKERS
    chunks_per_tile = rays_per_tile // chunk
    groups_per_chunk = chunk // LANES
    mesh = plsc.VectorSubcoreMesh(core_axis_name="c", subcore_axis_name="s")

    @functools.partial(
        pl.kernel,
        mesh=mesh,
        compiler_params=pltpu.CompilerParams(needs_layout_passes=False),
        out_type=jax.ShapeDtypeStruct((n,), jnp.float32),
        scratch_types=[
            pltpu.VMEM((N_ALPHA * N_DEPTH,), jnp.float32),
            pltpu.VMEM((chunk,), jnp.float32),
            pltpu.VMEM((chunk,), jnp.float32),
            pltpu.VMEM((chunk,), jnp.float32),
        ],
    )
    def sc_call(y_hbm, ld_hbm, cp_hbm, out_hbm, table_v, y_v, ld_v, out_v):
        wid = lax.axis_index("s") * 2 + lax.axis_index("c")
        base = wid * rays_per_tile
        pltpu.sync_copy(cp_hbm, table_v)

        def do_chunk(c, carry):
            row0 = base + c * chunk
            pltpu.sync_copy(y_hbm.at[pl.ds(row0, chunk)], y_v)
            pltpu.sync_copy(ld_hbm.at[pl.ds(row0, chunk)], ld_v)

            @plsc.parallel_loop(0, groups_per_chunk, unroll=4)
            def do_group(g):
                off = g * LANES
                y = y_v[pl.ds(off, LANES)]
                ld = ld_v[pl.ds(off, LANES)]

                yc = jnp.clip(y, -1.0, 1.0)
                alpha = _vasin(yc)
                a_norm = jnp.clip((alpha - ALPHA_MIN) / ALPHA_RANGE, 0.0, 1.0)
                d_norm = jnp.clip((ld - LD_MIN) / LD_RANGE, 0.0, 1.0)
                a_idx = a_norm * (N_ALPHA - 1)
                d_idx = d_norm * (N_DEPTH - 1)
                fa = a_idx.astype(jnp.int32)
                fd = d_idx.astype(jnp.int32)
                a_loc = a_idx - fa.astype(jnp.float32)
                d_loc = d_idx - fd.astype(jnp.float32)
                a_start = jnp.clip(fa - 1, 0, N_ALPHA - 4)
                d_start = jnp.clip(fd - 1, 0, N_DEPTH - 4)

                ab = _bspline4(a_loc)
                db = _bspline4(d_loc)

                flat0 = a_start * N_DEPTH + d_start
                acc = None
                for i in range(4):
                    row = None
                    for j in range(4):
                        cv = plsc.load_gather(
                            table_v, [flat0 + (i * N_DEPTH + j)])
                        term = db[j] * cv
                        row = term if row is None else row + term
                    term = ab[i] * row
                    acc = term if acc is None else acc + term

                res = jnp.clip(acc, -MAX_DELTA, MAX_DELTA)
                out_v[pl.ds(off, LANES)] = res

            pltpu.sync_copy(out_v, out_hbm.at[pl.ds(row0, chunk)])
            return carry

        lax.fori_loop(0, chunks_per_tile, do_chunk, 0)

    return sc_call


def kernel(ray_dirs, log_depth, control_points):
    n = ray_dirs.shape[0]
    ray_y = lax.squeeze(lax.slice(ray_dirs, (0, 1), (n, 2)), (1,))
    cp_flat = control_points.reshape(-1)
    sc_call = _make_sc_call(n, 4096)
    return sc_call(ray_y, log_depth, cp_flat)


# speedup vs baseline: 2015.0854x; 1.0509x over previous
import functools

import jax
import jax.numpy as jnp
from jax import lax
from jax.experimental import pallas as pl
from jax.experimental.pallas import tpu as pltpu
from jax.experimental.pallas import tpu_sc as plsc

N_ALPHA = 16
N_DEPTH = 12
ALPHA_MIN = -1.5707963267948966
ALPHA_RANGE = 3.141592653589793 + 1e-8
LD_MIN = -3.0
LD_RANGE = 8.0 + 1e-8
MAX_DELTA = 0.5

NUM_WORKERS = 32
LANES = 16

_ASIN_C = (
    1.5707963050,
    -0.2145988016,
    0.0889789874,
    -0.0501743046,
    0.0308918810,
    -0.0170881256,
    0.0066700901,
    -0.0012624911,
)


def _vsqrt(u):
    i = plsc.bitcast(u, jnp.int32)
    i = jnp.int32(0x5F3759DF) - lax.shift_right_logical(i, 1)
    r = plsc.bitcast(i, jnp.float32)
    half_u = 0.5 * u
    for _ in range(2):
        r = r * (1.5 - half_u * r * r)
    return u * r


def _vasin(x):
    t = jnp.abs(x)
    u = jnp.maximum(1.0 - t, 1e-14)
    sq = _vsqrt(u)
    p = jnp.full_like(t, _ASIN_C[7])
    for c in (_ASIN_C[6], _ASIN_C[5], _ASIN_C[4], _ASIN_C[3],
              _ASIN_C[2], _ASIN_C[1], _ASIN_C[0]):
        p = p * t + c
    a = 1.5707963267948966 - sq * p
    a = jnp.where(t >= 1.0, jnp.float32(1.5707963267948966), a)
    return jnp.where(x < 0.0, -a, a)


def _bspline4(u):
    u2 = u * u
    u3 = u2 * u
    omu = 1.0 - u
    b0 = (omu * omu * omu) * (1.0 / 6.0)
    b1 = 0.5 * u3 - u2 + (2.0 / 3.0)
    b3 = u3 * (1.0 / 6.0)
    b2 = 1.0 - b0 - b1 - b3
    return b0, b1, b2, b3


def _make_sc_call(n, chunk):
    rays_per_tile = n // NUM_WORKERS
    chunks_per_tile = rays_per_tile // chunk
    groups_per_chunk = chunk // LANES
    mesh = plsc.VectorSubcoreMesh(core_axis_name="c", subcore_axis_name="s")

    @functools.partial(
        pl.kernel,
        mesh=mesh,
        compiler_params=pltpu.CompilerParams(needs_layout_passes=False),
        out_type=jax.ShapeDtypeStruct((n,), jnp.float32),
        scratch_types=[
            pltpu.VMEM((N_ALPHA * N_DEPTH,), jnp.float32),
            pltpu.VMEM((chunk,), jnp.float32),
            pltpu.VMEM((chunk,), jnp.float32),
            pltpu.VMEM((chunk,), jnp.float32),
        ],
    )
    def sc_call(y_hbm, ld_hbm, cp_hbm, out_hbm, table_v, y_v, ld_v, out_v):
        wid = lax.axis_index("s") * 2 + lax.axis_index("c")
        base = wid * rays_per_tile
        pltpu.sync_copy(cp_hbm, table_v)

        def do_chunk(c, carry):
            row0 = base + c * chunk
            pltpu.sync_copy(y_hbm.at[pl.ds(row0, chunk)], y_v)
            pltpu.sync_copy(ld_hbm.at[pl.ds(row0, chunk)], ld_v)

            @plsc.parallel_loop(0, groups_per_chunk, unroll=8)
            def do_group(g):
                off = g * LANES
                y = y_v[pl.ds(off, LANES)]
                ld = ld_v[pl.ds(off, LANES)]

                yc = jnp.clip(y, -1.0, 1.0)
                alpha = _vasin(yc)
                a_idx = jnp.clip((alpha - ALPHA_MIN) * ((N_ALPHA - 1) / ALPHA_RANGE),
                                 0.0, float(N_ALPHA - 1))
                d_idx = jnp.clip((ld - LD_MIN) * ((N_DEPTH - 1) / LD_RANGE),
                                 0.0, float(N_DEPTH - 1))
                fa = a_idx.astype(jnp.int32)
                fd = d_idx.astype(jnp.int32)
                a_loc = a_idx - fa.astype(jnp.float32)
                d_loc = d_idx - fd.astype(jnp.float32)
                a_start = jnp.clip(fa - 1, 0, N_ALPHA - 4)
                d_start = jnp.clip(fd - 1, 0, N_DEPTH - 4)

                ab = _bspline4(a_loc)
                db = _bspline4(d_loc)

                flat0 = a_start * N_DEPTH + d_start
                acc = None
                for i in range(4):
                    row = None
                    for j in range(4):
                        cv = plsc.load_gather(
                            table_v, [flat0 + (i * N_DEPTH + j)])
                        term = db[j] * cv
                        row = term if row is None else row + term
                    term = ab[i] * row
                    acc = term if acc is None else acc + term

                res = jnp.clip(acc, -MAX_DELTA, MAX_DELTA)
                out_v[pl.ds(off, LANES)] = res

            pltpu.sync_copy(out_v, out_hbm.at[pl.ds(row0, chunk)])
            return carry

        lax.fori_loop(0, chunks_per_tile, do_chunk, 0)

    return sc_call


def kernel(ray_dirs, log_depth, control_points):
    n = ray_dirs.shape[0]
    ray_y = lax.squeeze(lax.slice(ray_dirs, (0, 1), (n, 2)), (1,))
    cp_flat = control_points.reshape(-1)
    sc_call = _make_sc_call(n, 8192)
    return sc_call(ray_y, log_depth, cp_flat)


# speedup vs baseline: 2234.9617x; 1.1091x over previous
import functools

import jax
import jax.numpy as jnp
from jax import lax
from jax.experimental import pallas as pl
from jax.experimental.pallas import tpu as pltpu
from jax.experimental.pallas import tpu_sc as plsc

N_ALPHA = 16
N_DEPTH = 12
ALPHA_MIN = -1.5707963267948966
ALPHA_RANGE = 3.141592653589793 + 1e-8
LD_MIN = -3.0
LD_RANGE = 8.0 + 1e-8
MAX_DELTA = 0.5

NUM_WORKERS = 32
LANES = 16

_ASIN_C = (
    1.5707963050,
    -0.2145988016,
    0.0889789874,
    -0.0501743046,
    0.0308918810,
    -0.0170881256,
    0.0066700901,
    -0.0012624911,
)


def _vsqrt(u):
    i = plsc.bitcast(u, jnp.int32)
    i = jnp.int32(0x5F3759DF) - lax.shift_right_logical(i, 1)
    r = plsc.bitcast(i, jnp.float32)
    half_u = 0.5 * u
    for _ in range(2):
        r = r * (1.5 - half_u * r * r)
    return u * r


def _vasin(x):
    t = jnp.abs(x)
    u = jnp.maximum(1.0 - t, 1e-14)
    sq = _vsqrt(u)
    p = jnp.full_like(t, _ASIN_C[7])
    for c in (_ASIN_C[6], _ASIN_C[5], _ASIN_C[4], _ASIN_C[3],
              _ASIN_C[2], _ASIN_C[1], _ASIN_C[0]):
        p = p * t + c
    a = 1.5707963267948966 - sq * p
    a = jnp.where(t >= 1.0, jnp.float32(1.5707963267948966), a)
    return jnp.where(x < 0.0, -a, a)


def _bspline4(u):
    u2 = u * u
    u3 = u2 * u
    omu = 1.0 - u
    b0 = (omu * omu * omu) * (1.0 / 6.0)
    b1 = 0.5 * u3 - u2 + (2.0 / 3.0)
    b3 = u3 * (1.0 / 6.0)
    b2 = 1.0 - b0 - b1 - b3
    return b0, b1, b2, b3


def _make_sc_call(n, chunk):
    rays_per_tile = n // NUM_WORKERS
    chunks_per_tile = rays_per_tile // chunk
    assert chunks_per_tile % 2 == 0
    n_pairs = chunks_per_tile // 2
    groups_per_chunk = chunk // LANES
    mesh = plsc.VectorSubcoreMesh(core_axis_name="c", subcore_axis_name="s")

    @functools.partial(
        pl.kernel,
        mesh=mesh,
        compiler_params=pltpu.CompilerParams(needs_layout_passes=False),
        out_type=jax.ShapeDtypeStruct((n,), jnp.float32),
        scratch_types=[
            pltpu.VMEM((N_ALPHA * N_DEPTH,), jnp.float32),
            pltpu.VMEM((2 * chunk,), jnp.float32),
            pltpu.VMEM((2 * chunk,), jnp.float32),
            pltpu.VMEM((2 * chunk,), jnp.float32),
            pltpu.SemaphoreType.DMA,
            pltpu.SemaphoreType.DMA,
            pltpu.SemaphoreType.DMA,
            pltpu.SemaphoreType.DMA,
        ],
    )
    def sc_call(y_hbm, ld_hbm, cp_hbm, out_hbm, table_v, y_v, ld_v, out_v,
                sem_in0, sem_in1, sem_out0, sem_out1):
        wid = lax.axis_index("s") * 2 + lax.axis_index("c")
        base = wid * rays_per_tile
        sem_in = (sem_in0, sem_in1)
        sem_out = (sem_out0, sem_out1)
        pltpu.sync_copy(cp_hbm, table_v)

        def in_copies(c, b):
            row0 = base + c * chunk
            return (
                pltpu.make_async_copy(
                    y_hbm.at[pl.ds(row0, chunk)],
                    y_v.at[pl.ds(b * chunk, chunk)], sem_in[b]),
                pltpu.make_async_copy(
                    ld_hbm.at[pl.ds(row0, chunk)],
                    ld_v.at[pl.ds(b * chunk, chunk)], sem_in[b]),
            )

        def out_copy(c, b):
            row0 = base + c * chunk
            return pltpu.make_async_copy(
                out_v.at[pl.ds(b * chunk, chunk)],
                out_hbm.at[pl.ds(row0, chunk)], sem_out[b])

        for b in range(2):
            for cp in in_copies(b, b):
                cp.start()

        def do_pair(i, carry):
            for b in range(2):
                c = 2 * i + b
                for cp in in_copies(c, b):
                    cp.wait()

                @pl.when(i > 0)
                def _wait_prev_scatter():
                    out_copy(c - 2, b).wait()

                boff = b * chunk

                @plsc.parallel_loop(0, groups_per_chunk, unroll=8)
                def do_group(g):
                    off = boff + g * LANES
                    y = y_v[pl.ds(off, LANES)]
                    ld = ld_v[pl.ds(off, LANES)]

                    yc = jnp.clip(y, -1.0, 1.0)
                    alpha = _vasin(yc)
                    a_idx = jnp.clip(
                        (alpha - ALPHA_MIN) * ((N_ALPHA - 1) / ALPHA_RANGE),
                        0.0, float(N_ALPHA - 1))
                    d_idx = jnp.clip(
                        (ld - LD_MIN) * ((N_DEPTH - 1) / LD_RANGE),
                        0.0, float(N_DEPTH - 1))
                    fa = a_idx.astype(jnp.int32)
                    fd = d_idx.astype(jnp.int32)
                    a_loc = a_idx - fa.astype(jnp.float32)
                    d_loc = d_idx - fd.astype(jnp.float32)
                    a_start = jnp.clip(fa - 1, 0, N_ALPHA - 4)
                    d_start = jnp.clip(fd - 1, 0, N_DEPTH - 4)

                    ab = _bspline4(a_loc)
                    db = _bspline4(d_loc)

                    flat0 = a_start * N_DEPTH + d_start
                    acc = None
                    for wi in range(4):
                        row = None
                        for wj in range(4):
                            cv = plsc.load_gather(
                                table_v, [flat0 + (wi * N_DEPTH + wj)])
                            term = db[wj] * cv
                            row = term if row is None else row + term
                        term = ab[wi] * row
                        acc = term if acc is None else acc + term

                    res = jnp.clip(acc, -MAX_DELTA, MAX_DELTA)
                    out_v[pl.ds(off, LANES)] = res

                @pl.when(i < n_pairs - 1)
                def _prefetch_next():
                    for cp in in_copies(c + 2, b):
                        cp.start()

                out_copy(c, b).start()
            return carry

        lax.fori_loop(0, n_pairs, do_pair, 0)
        for b in range(2):
            out_copy(chunks_per_tile - 2 + b, b).wait()

    return sc_call


def kernel(ray_dirs, log_depth, control_points):
    n = ray_dirs.shape[0]
    ray_y = lax.squeeze(lax.slice(ray_dirs, (0, 1), (n, 2)), (1,))
    cp_flat = control_points.reshape(-1)
    sc_call = _make_sc_call(n, 8192)
    return sc_call(ray_y, log_depth, cp_flat)
